# baseline (speedup 1.0000x reference)
import functools

import jax
import jax.numpy as jnp
from jax import lax
from jax.experimental import pallas as pl
from jax.experimental.pallas import tpu as pltpu
from jax.experimental.pallas import tpu_sc as plsc

_INFO = plsc.get_sparse_core_info()
_NC = _INFO.num_cores
_NS = _INFO.num_subcores
_NW = _NC * _NS


def _make_sc_gather(V, D, B):
    assert B % _NW == 0
    b_per_w = B // _NW
    mesh = plsc.VectorSubcoreMesh(core_axis_name="c", subcore_axis_name="s")

    @functools.partial(
        pl.kernel,
        mesh=mesh,
        out_type=(
            jax.ShapeDtypeStruct((B, D), jnp.float32),
            jax.ShapeDtypeStruct((B, D), jnp.float32),
        ),
        scratch_types=[
            pltpu.VMEM((b_per_w,), jnp.int32),
            pltpu.VMEM((b_per_w, D), jnp.float32),
            pltpu.SemaphoreType.DMA,
        ],
    )
    def sc_gather(table_hbm, idx1_hbm, idx2_hbm, e1_hbm, e2_hbm,
                  idx_v, rows_v, sem):
        wid = lax.axis_index("s") * _NC + lax.axis_index("c")
        base = wid * b_per_w
        pltpu.sync_copy(idx1_hbm.at[pl.ds(base, b_per_w)], idx_v)
        pltpu.async_copy(table_hbm.at[idx_v], rows_v, sem).wait()
        pltpu.sync_copy(rows_v, e1_hbm.at[pl.ds(base, b_per_w)])
        pltpu.sync_copy(idx2_hbm.at[pl.ds(base, b_per_w)], idx_v)
        pltpu.async_copy(table_hbm.at[idx_v], rows_v, sem).wait()
        pltpu.sync_copy(rows_v, e2_hbm.at[pl.ds(base, b_per_w)])

    return sc_gather


def _concat_t_body(alpha_ref, e1_ref, e2_ref, feat_ref, out_ref):
    a = alpha_ref[0, 0]
    out_ref[0, :, :] = a * e1_ref[...] + (1.0 - a) * e2_ref[...]
    out_ref[1:, :, :] = jnp.transpose(feat_ref[...], (1, 0, 2))


def kernel(feature, index_value_1, index_value_2, embedding_table, alpha):
    B, T, D = feature.shape
    V = embedding_table.shape[0]
    idx1 = index_value_1.astype(jnp.int32)
    idx2 = index_value_2.astype(jnp.int32)

    e1, e2 = _make_sc_gather(V, D, B)(embedding_table, idx1, idx2)

    BB = 32
    tposed = pl.pallas_call(
        _concat_t_body,
        grid=(B // BB,),
        in_specs=[
            pl.BlockSpec(memory_space=pltpu.SMEM),
            pl.BlockSpec((BB, D), lambda i: (i, 0)),
            pl.BlockSpec((BB, D), lambda i: (i, 0)),
            pl.BlockSpec((BB, T, D), lambda i: (i, 0, 0)),
        ],
        out_specs=pl.BlockSpec((T + 1, BB, D), lambda i: (0, i, 0)),
        out_shape=jax.ShapeDtypeStruct((T + 1, B, D), jnp.float32),
        compiler_params=pltpu.CompilerParams(
            dimension_semantics=("arbitrary",),
        ),
    )(jnp.reshape(alpha.astype(jnp.float32), (1, 1)), e1, e2, feature)
    return jnp.transpose(tposed, (1, 0, 2))


# speedup vs baseline: 1.0354x; 1.0354x over previous
import functools

import jax
import jax.numpy as jnp
from jax import lax
from jax.experimental import pallas as pl
from jax.experimental.pallas import tpu as pltpu
from jax.experimental.pallas import tpu_sc as plsc

_INFO = plsc.get_sparse_core_info()
_NC = _INFO.num_cores
_NS = _INFO.num_subcores
_NW = _NC * _NS


def _make_sc_gather(V, D, B):
    assert B % _NW == 0
    b_per_w = B // _NW
    mesh = plsc.VectorSubcoreMesh(core_axis_name="c", subcore_axis_name="s")

    @functools.partial(
        pl.kernel,
        mesh=mesh,
        out_type=(
            jax.ShapeDtypeStruct((B, D), jnp.float32),
            jax.ShapeDtypeStruct((B, D), jnp.float32),
        ),
        scratch_types=[
            pltpu.VMEM((b_per_w,), jnp.int32),
            pltpu.VMEM((b_per_w, D), jnp.float32),
            pltpu.SemaphoreType.DMA,
        ],
    )
    def sc_gather(table_hbm, idx1_hbm, idx2_hbm, e1_hbm, e2_hbm,
                  idx_v, rows_v, sem):
        wid = lax.axis_index("s") * _NC + lax.axis_index("c")
        base = wid * b_per_w
        pltpu.sync_copy(idx1_hbm.at[pl.ds(base, b_per_w)], idx_v)
        pltpu.async_copy(table_hbm.at[idx_v], rows_v, sem).wait()
        pltpu.sync_copy(rows_v, e1_hbm.at[pl.ds(base, b_per_w)])
        pltpu.sync_copy(idx2_hbm.at[pl.ds(base, b_per_w)], idx_v)
        pltpu.async_copy(table_hbm.at[idx_v], rows_v, sem).wait()
        pltpu.sync_copy(rows_v, e2_hbm.at[pl.ds(base, b_per_w)])

    return sc_gather


def _bulk_t_body(feat_ref, out_ref):
    out_ref[1:, :, :] = jnp.transpose(feat_ref[...], (1, 0, 2))


def _insert_body(alpha_ref, e1_ref, e2_ref, buf_ref, out_ref):
    a = alpha_ref[0, 0]
    out_ref[0, :, :] = a * e1_ref[...] + (1.0 - a) * e2_ref[...]


def kernel(feature, index_value_1, index_value_2, embedding_table, alpha):
    B, T, D = feature.shape
    V = embedding_table.shape[0]
    idx1 = index_value_1.astype(jnp.int32)
    idx2 = index_value_2.astype(jnp.int32)

    e1, e2 = _make_sc_gather(V, D, B)(embedding_table, idx1, idx2)

    BB = 32
    bulk = pl.pallas_call(
        _bulk_t_body,
        grid=(B // BB,),
        in_specs=[pl.BlockSpec((BB, T, D), lambda i: (i, 0, 0))],
        out_specs=pl.BlockSpec((T + 1, BB, D), lambda i: (0, i, 0)),
        out_shape=jax.ShapeDtypeStruct((T + 1, B, D), jnp.float32),
        compiler_params=pltpu.CompilerParams(
            dimension_semantics=("arbitrary",),
        ),
    )(feature)

    tposed = pl.pallas_call(
        _insert_body,
        grid=(1,),
        in_specs=[
            pl.BlockSpec(memory_space=pltpu.SMEM),
            pl.BlockSpec((B, D), lambda i: (0, 0)),
            pl.BlockSpec((B, D), lambda i: (0, 0)),
            pl.BlockSpec(memory_space=pltpu.MemorySpace.HBM),
        ],
        out_specs=pl.BlockSpec((1, B, D), lambda i: (0, 0, 0)),
        out_shape=jax.ShapeDtypeStruct((T + 1, B, D), jnp.float32),
        input_output_aliases={3: 0},
    )(jnp.reshape(alpha.astype(jnp.float32), (1, 1)), e1, e2, bulk)
    return jnp.transpose(tposed, (1, 0, 2))


# speedup vs baseline: 1.0746x; 1.0379x over previous
import functools

import jax
import jax.numpy as jnp
from jax import lax
from jax.experimental import pallas as pl
from jax.experimental.pallas import tpu as pltpu
from jax.experimental.pallas import tpu_sc as plsc

_INFO = plsc.get_sparse_core_info()
_NC = _INFO.num_cores
_NS = _INFO.num_subcores
_NW = _NC * _NS


def _make_sc_gather(V, D, B):
    assert B % _NW == 0
    b_per_w = B // _NW
    mesh = plsc.VectorSubcoreMesh(core_axis_name="c", subcore_axis_name="s")

    @functools.partial(
        pl.kernel,
        mesh=mesh,
        out_type=(
            jax.ShapeDtypeStruct((B, D), jnp.float32),
            jax.ShapeDtypeStruct((B, D), jnp.float32),
        ),
        scratch_types=[
            pltpu.VMEM((b_per_w,), jnp.int32),
            pltpu.VMEM((b_per_w, D), jnp.float32),
            pltpu.SemaphoreType.DMA,
        ],
    )
    def sc_gather(table_hbm, idx1_hbm, idx2_hbm, e1_hbm, e2_hbm,
                  idx_v, rows_v, sem):
        wid = lax.axis_index("s") * _NC + lax.axis_index("c")
        base = wid * b_per_w
        pltpu.sync_copy(idx1_hbm.at[pl.ds(base, b_per_w)], idx_v)
        pltpu.async_copy(table_hbm.at[idx_v], rows_v, sem).wait()
        pltpu.sync_copy(rows_v, e1_hbm.at[pl.ds(base, b_per_w)])
        pltpu.sync_copy(idx2_hbm.at[pl.ds(base, b_per_w)], idx_v)
        pltpu.async_copy(table_hbm.at[idx_v], rows_v, sem).wait()
        pltpu.sync_copy(rows_v, e2_hbm.at[pl.ds(base, b_per_w)])

    return sc_gather


def _bulk_t_body(feat_ref, out_ref):
    out_ref[1:, :, :] = jnp.transpose(feat_ref[...], (1, 0, 2))


def _insert_body(alpha_ref, e1_ref, e2_ref, buf_ref, out_ref):
    a = alpha_ref[0, 0]
    out_ref[0, :, :] = a * e1_ref[...] + (1.0 - a) * e2_ref[...]


def kernel(feature, index_value_1, index_value_2, embedding_table, alpha):
    B, T, D = feature.shape
    V = embedding_table.shape[0]
    idx1 = index_value_1.astype(jnp.int32)
    idx2 = index_value_2.astype(jnp.int32)

    e1, e2 = _make_sc_gather(V, D, B)(embedding_table, idx1, idx2)

    BB = 64
    bulk = pl.pallas_call(
        _bulk_t_body,
        grid=(B // BB,),
        in_specs=[pl.BlockSpec((BB, T, D), lambda i: (i, 0, 0))],
        out_specs=pl.BlockSpec((T + 1, BB, D), lambda i: (0, i, 0)),
        out_shape=jax.ShapeDtypeStruct((T + 1, B, D), jnp.float32),
        compiler_params=pltpu.CompilerParams(
            dimension_semantics=("arbitrary",),
        ),
    )(feature)

    tposed = pl.pallas_call(
        _insert_body,
        grid=(1,),
        in_specs=[
            pl.BlockSpec(memory_space=pltpu.SMEM),
            pl.BlockSpec((B, D), lambda i: (0, 0)),
            pl.BlockSpec((B, D), lambda i: (0, 0)),
            pl.BlockSpec(memory_space=pltpu.MemorySpace.HBM),
        ],
        out_specs=pl.BlockSpec((1, B, D), lambda i: (0, 0, 0)),
        out_shape=jax.ShapeDtypeStruct((T + 1, B, D), jnp.float32),
        input_output_aliases={3: 0},
    )(jnp.reshape(alpha.astype(jnp.float32), (1, 1)), e1, e2, bulk)
    return jnp.transpose(tposed, (1, 0, 2))


# speedup vs baseline: 1.0957x; 1.0196x over previous
import functools

import jax
import jax.numpy as jnp
from jax import lax
from jax.experimental import pallas as pl
from jax.experimental.pallas import tpu as pltpu
from jax.experimental.pallas import tpu_sc as plsc

_INFO = plsc.get_sparse_core_info()
_NC = _INFO.num_cores
_NS = _INFO.num_subcores
_NW = _NC * _NS


def _make_sc_gather(V, D, B):
    assert B % _NW == 0
    b_per_w = B // _NW
    mesh = plsc.VectorSubcoreMesh(core_axis_name="c", subcore_axis_name="s")

    @functools.partial(
        pl.kernel,
        mesh=mesh,
        out_type=(
            jax.ShapeDtypeStruct((B, D), jnp.float32),
            jax.ShapeDtypeStruct((B, D), jnp.float32),
        ),
        scratch_types=[
            pltpu.VMEM((b_per_w,), jnp.int32),
            pltpu.VMEM((b_per_w, D), jnp.float32),
            pltpu.SemaphoreType.DMA,
        ],
    )
    def sc_gather(table_hbm, idx1_hbm, idx2_hbm, e1_hbm, e2_hbm,
                  idx_v, rows_v, sem):
        wid = lax.axis_index("s") * _NC + lax.axis_index("c")
        base = wid * b_per_w
        pltpu.sync_copy(idx1_hbm.at[pl.ds(base, b_per_w)], idx_v)
        pltpu.async_copy(table_hbm.at[idx_v], rows_v, sem).wait()
        pltpu.sync_copy(rows_v, e1_hbm.at[pl.ds(base, b_per_w)])
        pltpu.sync_copy(idx2_hbm.at[pl.ds(base, b_per_w)], idx_v)
        pltpu.async_copy(table_hbm.at[idx_v], rows_v, sem).wait()
        pltpu.sync_copy(rows_v, e2_hbm.at[pl.ds(base, b_per_w)])

    return sc_gather


def _bulk_t_body(feat_ref, out_ref):
    out_ref[1:, :, :] = jnp.transpose(feat_ref[...], (1, 0, 2))


def _insert_body(alpha_ref, e1_ref, e2_ref, buf_ref, out_ref):
    a = alpha_ref[0, 0]
    out_ref[0, :, :] = a * e1_ref[...] + (1.0 - a) * e2_ref[...]


def kernel(feature, index_value_1, index_value_2, embedding_table, alpha):
    B, T, D = feature.shape
    V = embedding_table.shape[0]
    idx1 = index_value_1.astype(jnp.int32)
    idx2 = index_value_2.astype(jnp.int32)

    e1, e2 = _make_sc_gather(V, D, B)(embedding_table, idx1, idx2)

    BB = 128
    bulk = pl.pallas_call(
        _bulk_t_body,
        grid=(B // BB,),
        in_specs=[pl.BlockSpec((BB, T, D), lambda i: (i, 0, 0))],
        out_specs=pl.BlockSpec((T + 1, BB, D), lambda i: (0, i, 0)),
        out_shape=jax.ShapeDtypeStruct((T + 1, B, D), jnp.float32),
        compiler_params=pltpu.CompilerParams(
            dimension_semantics=("arbitrary",),
        ),
    )(feature)

    tposed = pl.pallas_call(
        _insert_body,
        grid=(1,),
        in_specs=[
            pl.BlockSpec(memory_space=pltpu.SMEM),
            pl.BlockSpec((B, D), lambda i: (0, 0)),
            pl.BlockSpec((B, D), lambda i: (0, 0)),
            pl.BlockSpec(memory_space=pltpu.MemorySpace.HBM),
        ],
        out_specs=pl.BlockSpec((1, B, D), lambda i: (0, 0, 0)),
        out_shape=jax.ShapeDtypeStruct((T + 1, B, D), jnp.float32),
        input_output_aliases={3: 0},
    )(jnp.reshape(alpha.astype(jnp.float32), (1, 1)), e1, e2, bulk)
    return jnp.transpose(tposed, (1, 0, 2))


# speedup vs baseline: 1.0999x; 1.0038x over previous
import functools

import jax
import jax.numpy as jnp
from jax import lax
from jax.experimental import pallas as pl
from jax.experimental.pallas import tpu as pltpu
from jax.experimental.pallas import tpu_sc as plsc

_INFO = plsc.get_sparse_core_info()
_NC = _INFO.num_cores
_NS = _INFO.num_subcores
_NW = _NC * _NS


def _make_sc_gather(V, D, B):
    assert B % _NW == 0
    b_per_w = B // _NW
    mesh = plsc.VectorSubcoreMesh(core_axis_name="c", subcore_axis_name="s")

    @functools.partial(
        pl.kernel,
        mesh=mesh,
        out_type=(
            jax.ShapeDtypeStruct((B, D), jnp.float32),
            jax.ShapeDtypeStruct((B, D), jnp.float32),
        ),
        scratch_types=[
            pltpu.VMEM((b_per_w,), jnp.int32),
            pltpu.VMEM((b_per_w, D), jnp.float32),
            pltpu.SemaphoreType.DMA,
        ],
    )
    def sc_gather(table_hbm, idx1_hbm, idx2_hbm, e1_hbm, e2_hbm,
                  idx_v, rows_v, sem):
        wid = lax.axis_index("s") * _NC + lax.axis_index("c")
        base = wid * b_per_w
        pltpu.sync_copy(idx1_hbm.at[pl.ds(base, b_per_w)], idx_v)
        pltpu.async_copy(table_hbm.at[idx_v], rows_v, sem).wait()
        pltpu.sync_copy(rows_v, e1_hbm.at[pl.ds(base, b_per_w)])
        pltpu.sync_copy(idx2_hbm.at[pl.ds(base, b_per_w)], idx_v)
        pltpu.async_copy(table_hbm.at[idx_v], rows_v, sem).wait()
        pltpu.sync_copy(rows_v, e2_hbm.at[pl.ds(base, b_per_w)])

    return sc_gather


_BBR = 128
_NBR = 2


def _bulk_ring_body(feat_ref, out_ref, inb0, inb1, outb0, outb1,
                    isems, osems):
    B = feat_ref.shape[0]
    T = feat_ref.shape[1]
    inbufs = (inb0, inb1)
    outbufs = (outb0, outb1)
    nblk = B // _BBR

    def in_copy(g):
        return pltpu.make_async_copy(
            feat_ref.at[pl.ds(g * _BBR, _BBR)], inbufs[g % _NBR],
            isems.at[g % _NBR])

    def out_copy(g):
        return pltpu.make_async_copy(
            outbufs[g % _NBR],
            out_ref.at[pl.ds(1, T), pl.ds(g * _BBR, _BBR), :],
            osems.at[g % _NBR])

    for g in range(_NBR):
        in_copy(g).start(priority=0)
    for g in range(nblk):
        s = g % _NBR
        if g >= _NBR:
            out_copy(g - _NBR).wait()
        in_copy(g).wait()
        outbufs[s][...] = jnp.transpose(inbufs[s][...], (1, 0, 2))
        out_copy(g).start(priority=1)
        if g + _NBR < nblk:
            in_copy(g + _NBR).start(priority=0)
    for g in range(max(nblk - _NBR, 0), nblk):
        out_copy(g).wait()


def _insert_body(alpha_ref, e1_ref, e2_ref, buf_ref, out_ref):
    a = alpha_ref[0, 0]
    out_ref[0, :, :] = a * e1_ref[...] + (1.0 - a) * e2_ref[...]


def kernel(feature, index_value_1, index_value_2, embedding_table, alpha):
    B, T, D = feature.shape
    V = embedding_table.shape[0]
    idx1 = index_value_1.astype(jnp.int32)
    idx2 = index_value_2.astype(jnp.int32)

    e1, e2 = _make_sc_gather(V, D, B)(embedding_table, idx1, idx2)

    bulk = pl.pallas_call(
        _bulk_ring_body,
        in_specs=[pl.BlockSpec(memory_space=pltpu.MemorySpace.HBM)],
        out_specs=pl.BlockSpec(memory_space=pltpu.MemorySpace.HBM),
        out_shape=jax.ShapeDtypeStruct((T + 1, B, D), jnp.float32),
        scratch_shapes=[
            pltpu.VMEM((_BBR, T, D), jnp.float32),
            pltpu.VMEM((_BBR, T, D), jnp.float32),
            pltpu.VMEM((T, _BBR, D), jnp.float32),
            pltpu.VMEM((T, _BBR, D), jnp.float32),
            pltpu.SemaphoreType.DMA((_NBR,)),
            pltpu.SemaphoreType.DMA((_NBR,)),
        ],
    )(feature)

    tposed = pl.pallas_call(
        _insert_body,
        grid=(1,),
        in_specs=[
            pl.BlockSpec(memory_space=pltpu.SMEM),
            pl.BlockSpec((B, D), lambda i: (0, 0)),
            pl.BlockSpec((B, D), lambda i: (0, 0)),
            pl.BlockSpec(memory_space=pltpu.MemorySpace.HBM),
        ],
        out_specs=pl.BlockSpec((1, B, D), lambda i: (0, 0, 0)),
        out_shape=jax.ShapeDtypeStruct((T + 1, B, D), jnp.float32),
        input_output_aliases={3: 0},
    )(jnp.reshape(alpha.astype(jnp.float32), (1, 1)), e1, e2, bulk)
    return jnp.transpose(tposed, (1, 0, 2))
